# baseline (speedup 1.0000x reference)
import jax
import jax.numpy as jnp
from jax.experimental import pallas as pl
from jax.experimental.pallas import tpu as pltpu

N_R = (1024, 2048, 1536, 1024, 512)
OFF = (0, 1024, 3072, 4608, 5632)
TOT = 6144
D = 256
TILE_T = 512
BM = 128


def _level(x, diag, pairs, wa, wb, wc):
    diag_ranks = sorted(diag)
    n_tiles = TOT // TILE_T
    tstart = {r: OFF[r] // TILE_T for r in range(5)}
    pos = n_tiles
    dstart, dsteps = {}, {}
    for r in diag_ranks:
        dstart[r] = pos
        dsteps[r] = N_R[r] // BM
        pos += dsteps[r]
    pstart, psteps = {}, {}
    for i in range(4):
        pstart[i] = pos
        psteps[i] = N_R[i] // BM
        pos += psteps[i]
    total_steps = pos

    mats = []
    for r in diag_ranks:
        mats += [diag[r][0], diag[r][1]]
    for i in range(4):
        mats += [pairs[i][0], pairs[i][1]]
    wa_ranks = sorted(wa)
    wb_ranks = sorted(wb)
    wc_ranks = sorted(wc)
    weights = [wa[r][0] for r in wa_ranks] + [wb[r][0] for r in wb_ranks] \
        + [wc[r][0] for r in wc_ranks]

    def body(*refs):
        x_ref = refs[0]
        mat_refs = refs[1:1 + len(mats)]
        w_refs = refs[1 + len(mats):1 + len(mats) + len(weights)]
        out_ref = refs[-4]
        ha_ref, hb_ref, hc_ref = refs[-3], refs[-2], refs[-1]
        wa_refs = {r: w_refs[k] for k, r in enumerate(wa_ranks)}
        wb_refs = {r: w_refs[len(wa_ranks) + k] for k, r in enumerate(wb_ranks)}
        wc_refs = {r: w_refs[len(wa_ranks) + len(wb_ranks) + k]
                   for k, r in enumerate(wc_ranks)}
        t = pl.program_id(0)

        @pl.when(t == 0)
        def _():
            out_ref[...] = jnp.zeros((TOT, D), jnp.float32)

        @pl.when(t < n_tiles)
        def _():
            xt = x_ref[...].astype(jnp.bfloat16)
            row = t * TILE_T
            for r in range(5):
                in_rank = (t >= tstart[r]) & (t < tstart[r] + N_R[r] // TILE_T)

                @pl.when(in_rank)
                def _(r=r, xt=xt, row=row):
                    for role, store in ((wa, ha_ref), (wb, hb_ref), (wc, hc_ref)):
                        if r in role:
                            w, s = role[r]
                            wref = (wa_refs if role is wa else
                                    wb_refs if role is wb else wc_refs)[r]
                            h = jnp.dot(xt, wref[...].astype(jnp.bfloat16),
                                        preferred_element_type=jnp.float32)
                            store[pl.ds(row, TILE_T), :] = (h * s).astype(jnp.bfloat16)

        for k, r in enumerate(diag_ranks):
            in_phase = (t >= dstart[r]) & (t < dstart[r] + dsteps[r])

            @pl.when(in_phase)
            def _(k=k, r=r):
                a_ref, c_ref = mat_refs[2 * k], mat_refs[2 * k + 1]
                na = (a_ref[...] * c_ref[...]).astype(jnp.bfloat16)
                acc = jnp.dot(na, ha_ref[pl.ds(OFF[r], N_R[r]), :],
                              preferred_element_type=jnp.float32)
                j = t - dstart[r]
                out_ref[pl.ds(OFF[r] + j * BM, BM), :] += acc

        for i in range(4):
            in_phase = (t >= pstart[i]) & (t < pstart[i] + psteps[i])

            @pl.when(in_phase)
            def _(i=i):
                base = 2 * len(diag_ranks)
                u_ref, cu_ref = mat_refs[base + 2 * i], mat_refs[base + 2 * i + 1]
                na = (u_ref[...] * cu_ref[...]).astype(jnp.bfloat16)
                j = t - pstart[i]
                row = OFF[i] + j * BM
                fwd = jnp.dot(na, hb_ref[pl.ds(OFF[i + 1], N_R[i + 1]), :],
                              preferred_element_type=jnp.float32)
                out_ref[pl.ds(row, BM), :] += fwd
                rev = jax.lax.dot_general(
                    na, hc_ref[pl.ds(row, BM), :], (((0,), (0,)), ((), ())),
                    preferred_element_type=jnp.float32)
                out_ref[pl.ds(OFF[i + 1], N_R[i + 1]), :] += rev

    def clip_map(start, steps):
        return lambda t: (jnp.clip(t - start, 0, steps - 1), 0)

    in_specs = [pl.BlockSpec((TILE_T, D), clip_map(0, n_tiles))]
    for r in diag_ranks:
        m = clip_map(dstart[r], dsteps[r])
        in_specs += [pl.BlockSpec((BM, N_R[r]), m), pl.BlockSpec((BM, N_R[r]), m)]
    for i in range(4):
        m = clip_map(pstart[i], psteps[i])
        in_specs += [pl.BlockSpec((BM, N_R[i + 1]), m),
                     pl.BlockSpec((BM, N_R[i + 1]), m)]
    in_specs += [pl.BlockSpec((D, D), lambda t: (0, 0))] * len(weights)

    return pl.pallas_call(
        body,
        grid=(total_steps,),
        in_specs=in_specs,
        out_specs=pl.BlockSpec((TOT, D), lambda t: (0, 0)),
        out_shape=jax.ShapeDtypeStruct((TOT, D), jnp.float32),
        scratch_shapes=[pltpu.VMEM((TOT, D), jnp.bfloat16)] * 3,
        compiler_params=pltpu.CompilerParams(
            dimension_semantics=("arbitrary",)),
    )(x, *mats, *weights)


def kernel(x_0, x_1, x_2, x_3, x_4, adjacency_0, adjacency_1, adjacency_2, adjacency_3, adjacency_4, cci_0_to_0, cci_1_to_1, cci_2_to_2, cci_3_to_3, cci_4_to_4, incidence_0_1, cci_0_to_1, incidence_0_2, cci_0_to_2, incidence_0_3, cci_0_to_3, incidence_0_4, cci_0_to_4, incidence_1_2, cci_1_to_2, incidence_1_3, cci_1_to_3, incidence_1_4, cci_1_to_4, incidence_2_3, cci_2_to_3, incidence_2_4, cci_2_to_4, incidence_3_4, cci_3_to_4, w_hbs_0_l1, w_hbs_4_l1, ws_hbns_0_1_l1, wt_hbns_0_1_l1, ws_hbns_1_2_l1, wt_hbns_1_2_l1, ws_hbns_2_3_l1, wt_hbns_2_3_l1, ws_hbns_3_4_l1, wt_hbns_3_4_l1, w_hbs_0_l2, w_hbs_1_l2, w_hbs_2_l2, w_hbs_3_l2, w_hbs_4_l2, ws_hbns_0_1_l2, wt_hbns_0_1_l2, ws_hbns_1_2_l2, wt_hbns_1_2_l2, ws_hbns_2_3_l2, wt_hbns_2_3_l2, ws_hbns_3_4_l2, wt_hbns_3_4_l2):
    pairs = [(incidence_0_1, cci_0_to_1), (incidence_1_2, cci_1_to_2),
             (incidence_2_3, cci_2_to_3), (incidence_3_4, cci_3_to_4)]
    x_st = jnp.concatenate([x_0, x_1, x_2, x_3, x_4], axis=0)

    x1 = _level(
        x_st,
        diag={0: (adjacency_0, cci_0_to_0), 4: (adjacency_4, cci_4_to_4)},
        pairs=pairs,
        wa={0: (w_hbs_0_l1, 0.5), 4: (w_hbs_4_l1, 0.5)},
        wb={1: (ws_hbns_0_1_l1, 0.5), 2: (ws_hbns_1_2_l1, 0.5),
            3: (ws_hbns_2_3_l1, 0.5), 4: (ws_hbns_3_4_l1, 0.5)},
        wc={0: (wt_hbns_0_1_l1, 0.5), 1: (wt_hbns_1_2_l1, 0.5),
            2: (wt_hbns_2_3_l1, 0.5), 3: (wt_hbns_3_4_l1, 0.5)},
    )

    k_agg = (2.0, 3.0, 3.0, 3.0, 2.0)
    out_st = _level(
        x1,
        diag={0: (adjacency_0, cci_0_to_0), 1: (adjacency_1, cci_1_to_1),
              2: (adjacency_2, cci_2_to_2), 3: (adjacency_3, cci_3_to_3),
              4: (adjacency_4, cci_4_to_4)},
        pairs=pairs,
        wa={r: (w, 1.0 / k_agg[r]) for r, w in enumerate(
            (w_hbs_0_l2, w_hbs_1_l2, w_hbs_2_l2, w_hbs_3_l2, w_hbs_4_l2))},
        wb={r: (w, 1.0 / k_agg[r - 1]) for r, w in
            ((1, ws_hbns_0_1_l2), (2, ws_hbns_1_2_l2),
             (3, ws_hbns_2_3_l2), (4, ws_hbns_3_4_l2))},
        wc={r: (w, 1.0 / k_agg[r + 1]) for r, w in
            ((0, wt_hbns_0_1_l2), (1, wt_hbns_1_2_l2),
             (2, wt_hbns_2_3_l2), (3, wt_hbns_3_4_l2))},
    )
    return tuple(out_st[OFF[r]:OFF[r] + N_R[r], :] for r in range(5))


# speedup vs baseline: 1.0759x; 1.0759x over previous
import jax
import jax.numpy as jnp
from jax.experimental import pallas as pl
from jax.experimental.pallas import tpu as pltpu

N_R = (1024, 2048, 1536, 1024, 512)
OFF = (0, 1024, 3072, 4608, 5632)
TOT = 6144
D = 256
TILE_T = 512
BM_D = 128
BM = 256


def _level(x, diag, pairs, wa, wb, wc):
    diag_ranks = sorted(diag)
    n_tiles = TOT // TILE_T
    tstart = {r: OFF[r] // TILE_T for r in range(5)}
    pos = n_tiles
    dstart, dsteps = {}, {}
    for r in diag_ranks:
        dstart[r] = pos
        dsteps[r] = N_R[r] // BM_D
        pos += dsteps[r]
    pstart, psteps = {}, {}
    for i in range(4):
        pstart[i] = pos
        psteps[i] = N_R[i] // BM
        pos += psteps[i]
    total_steps = pos

    mats = []
    for r in diag_ranks:
        mats += [diag[r][0], diag[r][1]]
    for i in range(4):
        mats += [pairs[i][0], pairs[i][1]]
    wa_ranks = sorted(wa)
    wb_ranks = sorted(wb)
    wc_ranks = sorted(wc)
    weights = [wa[r][0].astype(jnp.bfloat16) for r in wa_ranks] \
        + [wb[r][0].astype(jnp.bfloat16) for r in wb_ranks] \
        + [wc[r][0].astype(jnp.bfloat16) for r in wc_ranks]
    x = x.astype(jnp.bfloat16)

    def body(*refs):
        x_ref = refs[0]
        mat_refs = refs[1:1 + len(mats)]
        w_refs = refs[1 + len(mats):1 + len(mats) + len(weights)]
        out_ref = refs[-4]
        ha_ref, hb_ref, hc_ref = refs[-3], refs[-2], refs[-1]
        wa_refs = {r: w_refs[k] for k, r in enumerate(wa_ranks)}
        wb_refs = {r: w_refs[len(wa_ranks) + k] for k, r in enumerate(wb_ranks)}
        wc_refs = {r: w_refs[len(wa_ranks) + len(wb_ranks) + k]
                   for k, r in enumerate(wc_ranks)}
        t = pl.program_id(0)

        @pl.when(t == 0)
        def _():
            out_ref[...] = jnp.zeros((TOT, D), jnp.float32)

        @pl.when(t < n_tiles)
        def _():
            xt = x_ref[...]
            row = t * TILE_T
            for r in range(5):
                in_rank = (t >= tstart[r]) & (t < tstart[r] + N_R[r] // TILE_T)

                @pl.when(in_rank)
                def _(r=r, xt=xt, row=row):
                    for role, store in ((wa, ha_ref), (wb, hb_ref), (wc, hc_ref)):
                        if r in role:
                            w, s = role[r]
                            wref = (wa_refs if role is wa else
                                    wb_refs if role is wb else wc_refs)[r]
                            h = jnp.dot(xt, wref[...],
                                        preferred_element_type=jnp.float32)
                            store[pl.ds(row, TILE_T), :] = (h * s).astype(jnp.bfloat16)

        for k, r in enumerate(diag_ranks):
            in_phase = (t >= dstart[r]) & (t < dstart[r] + dsteps[r])

            @pl.when(in_phase)
            def _(k=k, r=r):
                a_ref, c_ref = mat_refs[2 * k], mat_refs[2 * k + 1]
                na = (a_ref[...] * c_ref[...]).astype(jnp.bfloat16)
                acc = jnp.dot(na, ha_ref[pl.ds(OFF[r], N_R[r]), :],
                              preferred_element_type=jnp.float32)
                j = t - dstart[r]
                out_ref[pl.ds(OFF[r] + j * BM_D, BM_D), :] += acc

        for i in range(4):
            in_phase = (t >= pstart[i]) & (t < pstart[i] + psteps[i])

            @pl.when(in_phase)
            def _(i=i):
                base = 2 * len(diag_ranks)
                u_ref, cu_ref = mat_refs[base + 2 * i], mat_refs[base + 2 * i + 1]
                na = (u_ref[...] * cu_ref[...]).astype(jnp.bfloat16)
                j = t - pstart[i]
                row = OFF[i] + j * BM
                fwd = jnp.dot(na, hb_ref[pl.ds(OFF[i + 1], N_R[i + 1]), :],
                              preferred_element_type=jnp.float32)
                out_ref[pl.ds(row, BM), :] += fwd
                rev = jax.lax.dot_general(
                    na, hc_ref[pl.ds(row, BM), :], (((0,), (0,)), ((), ())),
                    preferred_element_type=jnp.float32)
                out_ref[pl.ds(OFF[i + 1], N_R[i + 1]), :] += rev

    def clip_map(start, steps):
        return lambda t: (jnp.clip(t - start, 0, steps - 1), 0)

    in_specs = [pl.BlockSpec((TILE_T, D), clip_map(0, n_tiles))]
    for r in diag_ranks:
        m = clip_map(dstart[r], dsteps[r])
        in_specs += [pl.BlockSpec((BM_D, N_R[r]), m),
                     pl.BlockSpec((BM_D, N_R[r]), m)]
    for i in range(4):
        m = clip_map(pstart[i], psteps[i])
        in_specs += [pl.BlockSpec((BM, N_R[i + 1]), m),
                     pl.BlockSpec((BM, N_R[i + 1]), m)]
    in_specs += [pl.BlockSpec((D, D), lambda t: (0, 0))] * len(weights)

    return pl.pallas_call(
        body,
        grid=(total_steps,),
        in_specs=in_specs,
        out_specs=pl.BlockSpec((TOT, D), lambda t: (0, 0)),
        out_shape=jax.ShapeDtypeStruct((TOT, D), jnp.float32),
        scratch_shapes=[pltpu.VMEM((TOT, D), jnp.bfloat16)] * 3,
        compiler_params=pltpu.CompilerParams(
            dimension_semantics=("arbitrary",)),
    )(x, *mats, *weights)


def kernel(x_0, x_1, x_2, x_3, x_4, adjacency_0, adjacency_1, adjacency_2, adjacency_3, adjacency_4, cci_0_to_0, cci_1_to_1, cci_2_to_2, cci_3_to_3, cci_4_to_4, incidence_0_1, cci_0_to_1, incidence_0_2, cci_0_to_2, incidence_0_3, cci_0_to_3, incidence_0_4, cci_0_to_4, incidence_1_2, cci_1_to_2, incidence_1_3, cci_1_to_3, incidence_1_4, cci_1_to_4, incidence_2_3, cci_2_to_3, incidence_2_4, cci_2_to_4, incidence_3_4, cci_3_to_4, w_hbs_0_l1, w_hbs_4_l1, ws_hbns_0_1_l1, wt_hbns_0_1_l1, ws_hbns_1_2_l1, wt_hbns_1_2_l1, ws_hbns_2_3_l1, wt_hbns_2_3_l1, ws_hbns_3_4_l1, wt_hbns_3_4_l1, w_hbs_0_l2, w_hbs_1_l2, w_hbs_2_l2, w_hbs_3_l2, w_hbs_4_l2, ws_hbns_0_1_l2, wt_hbns_0_1_l2, ws_hbns_1_2_l2, wt_hbns_1_2_l2, ws_hbns_2_3_l2, wt_hbns_2_3_l2, ws_hbns_3_4_l2, wt_hbns_3_4_l2):
    pairs = [(incidence_0_1, cci_0_to_1), (incidence_1_2, cci_1_to_2),
             (incidence_2_3, cci_2_to_3), (incidence_3_4, cci_3_to_4)]
    x_st = jnp.concatenate([x_0, x_1, x_2, x_3, x_4], axis=0)

    x1 = _level(
        x_st,
        diag={0: (adjacency_0, cci_0_to_0), 4: (adjacency_4, cci_4_to_4)},
        pairs=pairs,
        wa={0: (w_hbs_0_l1, 0.5), 4: (w_hbs_4_l1, 0.5)},
        wb={1: (ws_hbns_0_1_l1, 0.5), 2: (ws_hbns_1_2_l1, 0.5),
            3: (ws_hbns_2_3_l1, 0.5), 4: (ws_hbns_3_4_l1, 0.5)},
        wc={0: (wt_hbns_0_1_l1, 0.5), 1: (wt_hbns_1_2_l1, 0.5),
            2: (wt_hbns_2_3_l1, 0.5), 3: (wt_hbns_3_4_l1, 0.5)},
    )

    k_agg = (2.0, 3.0, 3.0, 3.0, 2.0)
    out_st = _level(
        x1,
        diag={0: (adjacency_0, cci_0_to_0), 1: (adjacency_1, cci_1_to_1),
              2: (adjacency_2, cci_2_to_2), 3: (adjacency_3, cci_3_to_3),
              4: (adjacency_4, cci_4_to_4)},
        pairs=pairs,
        wa={r: (w, 1.0 / k_agg[r]) for r, w in enumerate(
            (w_hbs_0_l2, w_hbs_1_l2, w_hbs_2_l2, w_hbs_3_l2, w_hbs_4_l2))},
        wb={r: (w, 1.0 / k_agg[r - 1]) for r, w in
            ((1, ws_hbns_0_1_l2), (2, ws_hbns_1_2_l2),
             (3, ws_hbns_2_3_l2), (4, ws_hbns_3_4_l2))},
        wc={r: (w, 1.0 / k_agg[r + 1]) for r, w in
            ((0, wt_hbns_0_1_l2), (1, wt_hbns_1_2_l2),
             (2, wt_hbns_2_3_l2), (3, wt_hbns_3_4_l2))},
    )
    return tuple(out_st[OFF[r]:OFF[r] + N_R[r], :] for r in range(5))


# speedup vs baseline: 1.1774x; 1.0943x over previous
import functools

import jax
import jax.numpy as jnp
from jax.experimental import pallas as pl
from jax.experimental.pallas import tpu as pltpu

D = 256


def _transform_body(nparts, scales, *refs):
    nw = len(scales)
    parts = refs[:nparts]
    ws = refs[nparts:nparts + nw]
    outs = refs[nparts + nw:]
    x = parts[0][...]
    for p in parts[1:]:
        x = x + p[...]
    xb = x.astype(jnp.bfloat16)
    for w_ref, o_ref, s in zip(ws, outs, scales):
        h = jnp.dot(xb, w_ref[...], preferred_element_type=jnp.float32)
        o_ref[...] = (h * s).astype(jnp.bfloat16)


def _transform(parts, ws, scales, bm=512):
    n = parts[0].shape[0]
    bm = min(bm, n)
    ws = [w.astype(jnp.bfloat16) for w in ws]
    in_specs = [pl.BlockSpec((bm, D), lambda i: (i, 0)) for _ in parts]
    in_specs += [pl.BlockSpec((D, D), lambda i: (0, 0)) for _ in ws]
    out_specs = [pl.BlockSpec((bm, D), lambda i: (i, 0)) for _ in ws]
    return pl.pallas_call(
        functools.partial(_transform_body, len(parts), tuple(scales)),
        grid=(n // bm,),
        in_specs=in_specs,
        out_specs=out_specs,
        out_shape=[jax.ShapeDtypeStruct((n, D), jnp.bfloat16) for _ in ws],
    )(*parts, *ws)


def _hbs_body(has_acc, *refs):
    if has_acc:
        a_ref, c_ref, h_ref, acc_ref, o_ref = refs
    else:
        a_ref, c_ref, h_ref, o_ref = refs
    na = (a_ref[...] * c_ref[...]).astype(jnp.bfloat16)
    r = jnp.dot(na, h_ref[...], preferred_element_type=jnp.float32)
    if has_acc:
        r = r + acc_ref[...]
    o_ref[...] = r


def _hbs(a, c, h, acc=None, bm=512):
    m, k = a.shape
    bm = min(bm, m)
    in_specs = [
        pl.BlockSpec((bm, k), lambda i: (i, 0)),
        pl.BlockSpec((bm, k), lambda i: (i, 0)),
        pl.BlockSpec((k, D), lambda i: (0, 0)),
    ]
    args = [a, c, h]
    aliases = {}
    if acc is not None:
        in_specs.append(pl.BlockSpec((bm, D), lambda i: (i, 0)))
        args.append(acc)
        aliases = {3: 0}
    return pl.pallas_call(
        functools.partial(_hbs_body, acc is not None),
        grid=(m // bm,),
        in_specs=in_specs,
        out_specs=pl.BlockSpec((bm, D), lambda i: (i, 0)),
        out_shape=jax.ShapeDtypeStruct((m, D), jnp.float32),
        input_output_aliases=aliases,
    )(*args)


def _dual_body(acc_t, acc_s, *refs):
    refs = list(refs)
    a_ref = refs.pop(0)
    c_ref = refs.pop(0)
    hs_ref = refs.pop(0)
    ht_ref = refs.pop(0)
    at_ref = refs.pop(0) if acc_t else None
    as_ref = refs.pop(0) if acc_s else None
    ot_ref, os_ref = refs
    i = pl.program_id(0)
    na = (a_ref[...] * c_ref[...]).astype(jnp.bfloat16)
    fwd = jnp.dot(na, hs_ref[...], preferred_element_type=jnp.float32)
    if acc_t:
        fwd = fwd + at_ref[...]
    ot_ref[...] = fwd
    rev = jax.lax.dot_general(
        na, ht_ref[...], (((0,), (0,)), ((), ())),
        preferred_element_type=jnp.float32)

    @pl.when(i == 0)
    def _():
        os_ref[...] = (rev + as_ref[...]) if acc_s else rev

    @pl.when(i > 0)
    def _():
        os_ref[...] += rev


def _dual(a, c, hs, ht, acc_t=None, acc_s=None, bm=512):
    m, k = a.shape
    bm = min(bm, m)
    in_specs = [
        pl.BlockSpec((bm, k), lambda i: (i, 0)),
        pl.BlockSpec((bm, k), lambda i: (i, 0)),
        pl.BlockSpec((k, D), lambda i: (0, 0)),
        pl.BlockSpec((bm, D), lambda i: (i, 0)),
    ]
    args = [a, c, hs, ht]
    aliases = {}
    if acc_t is not None:
        in_specs.append(pl.BlockSpec((bm, D), lambda i: (i, 0)))
        args.append(acc_t)
        aliases[len(args) - 1] = 0
    if acc_s is not None:
        in_specs.append(pl.BlockSpec((k, D), lambda i: (0, 0)))
        args.append(acc_s)
        aliases[len(args) - 1] = 1
    return pl.pallas_call(
        functools.partial(_dual_body, acc_t is not None, acc_s is not None),
        grid=(m // bm,),
        in_specs=in_specs,
        out_specs=[
            pl.BlockSpec((bm, D), lambda i: (i, 0)),
            pl.BlockSpec((k, D), lambda i: (0, 0)),
        ],
        out_shape=[
            jax.ShapeDtypeStruct((m, D), jnp.float32),
            jax.ShapeDtypeStruct((k, D), jnp.float32),
        ],
        input_output_aliases=aliases,
        compiler_params=pltpu.CompilerParams(
            dimension_semantics=("arbitrary",)),
    )(*args)


def kernel(x_0, x_1, x_2, x_3, x_4, adjacency_0, adjacency_1, adjacency_2, adjacency_3, adjacency_4, cci_0_to_0, cci_1_to_1, cci_2_to_2, cci_3_to_3, cci_4_to_4, incidence_0_1, cci_0_to_1, incidence_0_2, cci_0_to_2, incidence_0_3, cci_0_to_3, incidence_0_4, cci_0_to_4, incidence_1_2, cci_1_to_2, incidence_1_3, cci_1_to_3, incidence_1_4, cci_1_to_4, incidence_2_3, cci_2_to_3, incidence_2_4, cci_2_to_4, incidence_3_4, cci_3_to_4, w_hbs_0_l1, w_hbs_4_l1, ws_hbns_0_1_l1, wt_hbns_0_1_l1, ws_hbns_1_2_l1, wt_hbns_1_2_l1, ws_hbns_2_3_l1, wt_hbns_2_3_l1, ws_hbns_3_4_l1, wt_hbns_3_4_l1, w_hbs_0_l2, w_hbs_1_l2, w_hbs_2_l2, w_hbs_3_l2, w_hbs_4_l2, ws_hbns_0_1_l2, wt_hbns_0_1_l2, ws_hbns_1_2_l2, wt_hbns_1_2_l2, ws_hbns_2_3_l2, wt_hbns_2_3_l2, ws_hbns_3_4_l2, wt_hbns_3_4_l2):
    h0_hbs, ft01 = _transform([x_0], [w_hbs_0_l1, wt_hbns_0_1_l1], [1.0, 1.0])
    fs01, ft12 = _transform([x_1], [ws_hbns_0_1_l1, wt_hbns_1_2_l1], [1.0, 1.0])
    fs12, ft23 = _transform([x_2], [ws_hbns_1_2_l1, wt_hbns_2_3_l1], [1.0, 1.0])
    fs23, ft34 = _transform([x_3], [ws_hbns_2_3_l1, wt_hbns_3_4_l1], [1.0, 1.0])
    h4_hbs, fs34 = _transform([x_4], [w_hbs_4_l1, ws_hbns_3_4_l1], [1.0, 1.0])

    x0 = _hbs(adjacency_0, cci_0_to_0, h0_hbs)
    x0, x1 = _dual(incidence_0_1, cci_0_to_1, fs01, ft01, acc_t=x0)
    x1, x2 = _dual(incidence_1_2, cci_1_to_2, fs12, ft12, acc_t=x1)
    x2, x3 = _dual(incidence_2_3, cci_2_to_3, fs23, ft23, acc_t=x2)
    x3, x4 = _dual(incidence_3_4, cci_3_to_4, fs34, ft34, acc_t=x3)
    x4 = _hbs(adjacency_4, cci_4_to_4, h4_hbs, acc=x4)

    k_agg = (2.0, 3.0, 3.0, 3.0, 2.0)

    def s(tgt):
        return 0.5 / k_agg[tgt]

    h0b, ft01b = _transform([x0], [w_hbs_0_l2, wt_hbns_0_1_l2], [s(0), s(1)])
    h1b, fs01b, ft12b = _transform(
        [x1], [w_hbs_1_l2, ws_hbns_0_1_l2, wt_hbns_1_2_l2], [s(1), s(0), s(2)])
    h2b, fs12b, ft23b = _transform(
        [x2], [w_hbs_2_l2, ws_hbns_1_2_l2, wt_hbns_2_3_l2], [s(2), s(1), s(3)])
    h3b, fs23b, ft34b = _transform(
        [x3], [w_hbs_3_l2, ws_hbns_2_3_l2, wt_hbns_3_4_l2], [s(3), s(2), s(4)])
    h4b, fs34b = _transform([x4], [w_hbs_4_l2, ws_hbns_3_4_l2], [s(4), s(3)])

    out0 = _hbs(adjacency_0, cci_0_to_0, h0b)
    out1 = _hbs(adjacency_1, cci_1_to_1, h1b)
    out2 = _hbs(adjacency_2, cci_2_to_2, h2b)
    out3 = _hbs(adjacency_3, cci_3_to_3, h3b)
    out4 = _hbs(adjacency_4, cci_4_to_4, h4b)
    out0, out1 = _dual(incidence_0_1, cci_0_to_1, fs01b, ft01b,
                       acc_t=out0, acc_s=out1)
    out1, out2 = _dual(incidence_1_2, cci_1_to_2, fs12b, ft12b,
                       acc_t=out1, acc_s=out2)
    out2, out3 = _dual(incidence_2_3, cci_2_to_3, fs23b, ft23b,
                       acc_t=out2, acc_s=out3)
    out3, out4 = _dual(incidence_3_4, cci_3_to_4, fs34b, ft34b,
                       acc_t=out3, acc_s=out4)
    return (out0, out1, out2, out3, out4)


# speedup vs baseline: 1.1775x; 1.0001x over previous
import functools

import jax
import jax.numpy as jnp
from jax.experimental import pallas as pl
from jax.experimental.pallas import tpu as pltpu

D = 256


def _transform_body(nparts, scales, *refs):
    nw = len(scales)
    parts = refs[:nparts]
    ws = refs[nparts:nparts + nw]
    outs = refs[nparts + nw:]
    x = parts[0][...]
    for p in parts[1:]:
        x = x + p[...]
    xb = x.astype(jnp.bfloat16)
    for w_ref, o_ref, s in zip(ws, outs, scales):
        h = jnp.dot(xb, w_ref[...], preferred_element_type=jnp.float32)
        o_ref[...] = (h * s).astype(jnp.bfloat16)


def _transform(parts, ws, scales, bm=512):
    n = parts[0].shape[0]
    bm = min(bm, n)
    ws = [w.astype(jnp.bfloat16) for w in ws]
    in_specs = [pl.BlockSpec((bm, D), lambda i: (i, 0)) for _ in parts]
    in_specs += [pl.BlockSpec((D, D), lambda i: (0, 0)) for _ in ws]
    out_specs = [pl.BlockSpec((bm, D), lambda i: (i, 0)) for _ in ws]
    return pl.pallas_call(
        functools.partial(_transform_body, len(parts), tuple(scales)),
        grid=(n // bm,),
        in_specs=in_specs,
        out_specs=out_specs,
        out_shape=[jax.ShapeDtypeStruct((n, D), jnp.bfloat16) for _ in ws],
        compiler_params=pltpu.CompilerParams(
            dimension_semantics=("parallel",)),
    )(*parts, *ws)


def _hbs_body(has_acc, *refs):
    if has_acc:
        a_ref, c_ref, h_ref, acc_ref, o_ref = refs
    else:
        a_ref, c_ref, h_ref, o_ref = refs
    na = (a_ref[...] * c_ref[...]).astype(jnp.bfloat16)
    r = jnp.dot(na, h_ref[...], preferred_element_type=jnp.float32)
    if has_acc:
        r = r + acc_ref[...]
    o_ref[...] = r


def _hbs(a, c, h, acc=None, bm=512):
    m, k = a.shape
    bm = min(bm, m)
    in_specs = [
        pl.BlockSpec((bm, k), lambda i: (i, 0)),
        pl.BlockSpec((bm, k), lambda i: (i, 0)),
        pl.BlockSpec((k, D), lambda i: (0, 0)),
    ]
    args = [a, c, h]
    aliases = {}
    if acc is not None:
        in_specs.append(pl.BlockSpec((bm, D), lambda i: (i, 0)))
        args.append(acc)
        aliases = {3: 0}
    return pl.pallas_call(
        functools.partial(_hbs_body, acc is not None),
        grid=(m // bm,),
        in_specs=in_specs,
        out_specs=pl.BlockSpec((bm, D), lambda i: (i, 0)),
        out_shape=jax.ShapeDtypeStruct((m, D), jnp.float32),
        input_output_aliases=aliases,
        compiler_params=pltpu.CompilerParams(
            dimension_semantics=("parallel",)),
    )(*args)


def _dual_body(acc_t, acc_s, *refs):
    refs = list(refs)
    a_ref = refs.pop(0)
    c_ref = refs.pop(0)
    hs_ref = refs.pop(0)
    ht_ref = refs.pop(0)
    at_ref = refs.pop(0) if acc_t else None
    as_ref = refs.pop(0) if acc_s else None
    ot_ref, os_ref = refs
    i = pl.program_id(0)
    na = (a_ref[...] * c_ref[...]).astype(jnp.bfloat16)
    fwd = jnp.dot(na, hs_ref[...], preferred_element_type=jnp.float32)
    if acc_t:
        fwd = fwd + at_ref[...]
    ot_ref[...] = fwd
    rev = jax.lax.dot_general(
        na, ht_ref[...], (((0,), (0,)), ((), ())),
        preferred_element_type=jnp.float32)

    @pl.when(i == 0)
    def _():
        os_ref[...] = (rev + as_ref[...]) if acc_s else rev

    @pl.when(i > 0)
    def _():
        os_ref[...] += rev


def _dual(a, c, hs, ht, acc_t=None, acc_s=None, bm=512):
    m, k = a.shape
    bm = min(bm, m)
    in_specs = [
        pl.BlockSpec((bm, k), lambda i: (i, 0)),
        pl.BlockSpec((bm, k), lambda i: (i, 0)),
        pl.BlockSpec((k, D), lambda i: (0, 0)),
        pl.BlockSpec((bm, D), lambda i: (i, 0)),
    ]
    args = [a, c, hs, ht]
    aliases = {}
    if acc_t is not None:
        in_specs.append(pl.BlockSpec((bm, D), lambda i: (i, 0)))
        args.append(acc_t)
        aliases[len(args) - 1] = 0
    if acc_s is not None:
        in_specs.append(pl.BlockSpec((k, D), lambda i: (0, 0)))
        args.append(acc_s)
        aliases[len(args) - 1] = 1
    return pl.pallas_call(
        functools.partial(_dual_body, acc_t is not None, acc_s is not None),
        grid=(m // bm,),
        in_specs=in_specs,
        out_specs=[
            pl.BlockSpec((bm, D), lambda i: (i, 0)),
            pl.BlockSpec((k, D), lambda i: (0, 0)),
        ],
        out_shape=[
            jax.ShapeDtypeStruct((m, D), jnp.float32),
            jax.ShapeDtypeStruct((k, D), jnp.float32),
        ],
        input_output_aliases=aliases,
        compiler_params=pltpu.CompilerParams(
            dimension_semantics=("arbitrary",)),
    )(*args)


def kernel(x_0, x_1, x_2, x_3, x_4, adjacency_0, adjacency_1, adjacency_2, adjacency_3, adjacency_4, cci_0_to_0, cci_1_to_1, cci_2_to_2, cci_3_to_3, cci_4_to_4, incidence_0_1, cci_0_to_1, incidence_0_2, cci_0_to_2, incidence_0_3, cci_0_to_3, incidence_0_4, cci_0_to_4, incidence_1_2, cci_1_to_2, incidence_1_3, cci_1_to_3, incidence_1_4, cci_1_to_4, incidence_2_3, cci_2_to_3, incidence_2_4, cci_2_to_4, incidence_3_4, cci_3_to_4, w_hbs_0_l1, w_hbs_4_l1, ws_hbns_0_1_l1, wt_hbns_0_1_l1, ws_hbns_1_2_l1, wt_hbns_1_2_l1, ws_hbns_2_3_l1, wt_hbns_2_3_l1, ws_hbns_3_4_l1, wt_hbns_3_4_l1, w_hbs_0_l2, w_hbs_1_l2, w_hbs_2_l2, w_hbs_3_l2, w_hbs_4_l2, ws_hbns_0_1_l2, wt_hbns_0_1_l2, ws_hbns_1_2_l2, wt_hbns_1_2_l2, ws_hbns_2_3_l2, wt_hbns_2_3_l2, ws_hbns_3_4_l2, wt_hbns_3_4_l2):
    h0_hbs, ft01 = _transform([x_0], [w_hbs_0_l1, wt_hbns_0_1_l1], [1.0, 1.0])
    fs01, ft12 = _transform([x_1], [ws_hbns_0_1_l1, wt_hbns_1_2_l1], [1.0, 1.0])
    fs12, ft23 = _transform([x_2], [ws_hbns_1_2_l1, wt_hbns_2_3_l1], [1.0, 1.0])
    fs23, ft34 = _transform([x_3], [ws_hbns_2_3_l1, wt_hbns_3_4_l1], [1.0, 1.0])
    h4_hbs, fs34 = _transform([x_4], [w_hbs_4_l1, ws_hbns_3_4_l1], [1.0, 1.0])

    x0 = _hbs(adjacency_0, cci_0_to_0, h0_hbs)
    x0, x1 = _dual(incidence_0_1, cci_0_to_1, fs01, ft01, acc_t=x0)
    x1, x2 = _dual(incidence_1_2, cci_1_to_2, fs12, ft12, acc_t=x1)
    x2, x3 = _dual(incidence_2_3, cci_2_to_3, fs23, ft23, acc_t=x2)
    x3, x4 = _dual(incidence_3_4, cci_3_to_4, fs34, ft34, acc_t=x3)
    x4 = _hbs(adjacency_4, cci_4_to_4, h4_hbs, acc=x4)

    k_agg = (2.0, 3.0, 3.0, 3.0, 2.0)

    def s(tgt):
        return 0.5 / k_agg[tgt]

    h0b, ft01b = _transform([x0], [w_hbs_0_l2, wt_hbns_0_1_l2], [s(0), s(1)])
    h1b, fs01b, ft12b = _transform(
        [x1], [w_hbs_1_l2, ws_hbns_0_1_l2, wt_hbns_1_2_l2], [s(1), s(0), s(2)])
    h2b, fs12b, ft23b = _transform(
        [x2], [w_hbs_2_l2, ws_hbns_1_2_l2, wt_hbns_2_3_l2], [s(2), s(1), s(3)])
    h3b, fs23b, ft34b = _transform(
        [x3], [w_hbs_3_l2, ws_hbns_2_3_l2, wt_hbns_3_4_l2], [s(3), s(2), s(4)])
    h4b, fs34b = _transform([x4], [w_hbs_4_l2, ws_hbns_3_4_l2], [s(4), s(3)])

    out0 = _hbs(adjacency_0, cci_0_to_0, h0b)
    out1 = _hbs(adjacency_1, cci_1_to_1, h1b)
    out2 = _hbs(adjacency_2, cci_2_to_2, h2b)
    out3 = _hbs(adjacency_3, cci_3_to_3, h3b)
    out4 = _hbs(adjacency_4, cci_4_to_4, h4b)
    out0, out1 = _dual(incidence_0_1, cci_0_to_1, fs01b, ft01b,
                       acc_t=out0, acc_s=out1)
    out1, out2 = _dual(incidence_1_2, cci_1_to_2, fs12b, ft12b,
                       acc_t=out1, acc_s=out2)
    out2, out3 = _dual(incidence_2_3, cci_2_to_3, fs23b, ft23b,
                       acc_t=out2, acc_s=out3)
    out3, out4 = _dual(incidence_3_4, cci_3_to_4, fs34b, ft34b,
                       acc_t=out3, acc_s=out4)
    return (out0, out1, out2, out3, out4)


# speedup vs baseline: 1.2264x; 1.0415x over previous
import functools

import jax
import jax.numpy as jnp
from jax.experimental import pallas as pl
from jax.experimental.pallas import tpu as pltpu

D = 256


def _transform_body(nparts, scales, *refs):
    nw = len(scales)
    parts = refs[:nparts]
    ws = refs[nparts:nparts + nw]
    outs = refs[nparts + nw:]
    x = parts[0][...]
    for p in parts[1:]:
        x = x + p[...]
    xb = x.astype(jnp.bfloat16)
    for w_ref, o_ref, s in zip(ws, outs, scales):
        h = jnp.dot(xb, w_ref[...], preferred_element_type=jnp.float32)
        o_ref[...] = (h * s).astype(jnp.bfloat16)


def _transform(parts, ws, scales, bm=512):
    n = parts[0].shape[0]
    bm = min(bm, n)
    ws = [w.astype(jnp.bfloat16) for w in ws]
    in_specs = [pl.BlockSpec((bm, D), lambda i: (i, 0)) for _ in parts]
    in_specs += [pl.BlockSpec((D, D), lambda i: (0, 0)) for _ in ws]
    out_specs = [pl.BlockSpec((bm, D), lambda i: (i, 0)) for _ in ws]
    return pl.pallas_call(
        functools.partial(_transform_body, len(parts), tuple(scales)),
        grid=(n // bm,),
        in_specs=in_specs,
        out_specs=out_specs,
        out_shape=[jax.ShapeDtypeStruct((n, D), jnp.bfloat16) for _ in ws],
        compiler_params=pltpu.CompilerParams(
            dimension_semantics=("parallel",)),
    )(*parts, *ws)


def _hbs_body(has_acc, masked_in, emit_na, *refs):
    refs = list(refs)
    a_ref = refs.pop(0)
    c_ref = None if masked_in else refs.pop(0)
    h_ref = refs.pop(0)
    acc_ref = refs.pop(0) if has_acc else None
    o_ref = refs.pop(0)
    na_ref = refs.pop(0) if emit_na else None
    if masked_in:
        na = a_ref[...]
    else:
        na = (a_ref[...] * c_ref[...]).astype(jnp.bfloat16)
    r = jnp.dot(na, h_ref[...], preferred_element_type=jnp.float32)
    if has_acc:
        r = r + acc_ref[...]
    o_ref[...] = r
    if emit_na:
        na_ref[...] = na


def _hbs(a, c, h, acc=None, emit_na=False, bm=512):
    m, k = a.shape
    bm = min(bm, m)
    row_spec = pl.BlockSpec((bm, k), lambda i: (i, 0))
    in_specs = [row_spec]
    args = [a]
    if c is not None:
        in_specs.append(row_spec)
        args.append(c)
    in_specs.append(pl.BlockSpec((k, D), lambda i: (0, 0)))
    args.append(h)
    aliases = {}
    if acc is not None:
        in_specs.append(pl.BlockSpec((bm, D), lambda i: (i, 0)))
        args.append(acc)
        aliases = {len(args) - 1: 0}
    out_specs = [pl.BlockSpec((bm, D), lambda i: (i, 0))]
    out_shape = [jax.ShapeDtypeStruct((m, D), jnp.float32)]
    if emit_na:
        out_specs.append(row_spec)
        out_shape.append(jax.ShapeDtypeStruct((m, k), jnp.bfloat16))
    res = pl.pallas_call(
        functools.partial(_hbs_body, acc is not None, c is None, emit_na),
        grid=(m // bm,),
        in_specs=in_specs,
        out_specs=out_specs,
        out_shape=out_shape,
        input_output_aliases=aliases,
        compiler_params=pltpu.CompilerParams(
            dimension_semantics=("parallel",)),
    )(*args)
    return res if emit_na else res[0]


def _dual_body(acc_t, acc_s, masked_in, emit_na, *refs):
    refs = list(refs)
    a_ref = refs.pop(0)
    c_ref = None if masked_in else refs.pop(0)
    hs_ref = refs.pop(0)
    ht_ref = refs.pop(0)
    at_ref = refs.pop(0) if acc_t else None
    as_ref = refs.pop(0) if acc_s else None
    ot_ref = refs.pop(0)
    os_ref = refs.pop(0)
    na_ref = refs.pop(0) if emit_na else None
    i = pl.program_id(0)
    if masked_in:
        na = a_ref[...]
    else:
        na = (a_ref[...] * c_ref[...]).astype(jnp.bfloat16)
    if emit_na:
        na_ref[...] = na
    fwd = jnp.dot(na, hs_ref[...], preferred_element_type=jnp.float32)
    if acc_t:
        fwd = fwd + at_ref[...]
    ot_ref[...] = fwd
    rev = jax.lax.dot_general(
        na, ht_ref[...], (((0,), (0,)), ((), ())),
        preferred_element_type=jnp.float32)

    @pl.when(i == 0)
    def _():
        os_ref[...] = (rev + as_ref[...]) if acc_s else rev

    @pl.when(i > 0)
    def _():
        os_ref[...] += rev


def _dual(a, c, hs, ht, acc_t=None, acc_s=None, emit_na=False, bm=512):
    m, k = a.shape
    bm = min(bm, m)
    row_spec = pl.BlockSpec((bm, k), lambda i: (i, 0))
    in_specs = [row_spec]
    args = [a]
    if c is not None:
        in_specs.append(row_spec)
        args.append(c)
    in_specs += [
        pl.BlockSpec((k, D), lambda i: (0, 0)),
        pl.BlockSpec((bm, D), lambda i: (i, 0)),
    ]
    args += [hs, ht]
    aliases = {}
    if acc_t is not None:
        in_specs.append(pl.BlockSpec((bm, D), lambda i: (i, 0)))
        args.append(acc_t)
        aliases[len(args) - 1] = 0
    if acc_s is not None:
        in_specs.append(pl.BlockSpec((k, D), lambda i: (0, 0)))
        args.append(acc_s)
        aliases[len(args) - 1] = 1
    out_specs = [
        pl.BlockSpec((bm, D), lambda i: (i, 0)),
        pl.BlockSpec((k, D), lambda i: (0, 0)),
    ]
    out_shape = [
        jax.ShapeDtypeStruct((m, D), jnp.float32),
        jax.ShapeDtypeStruct((k, D), jnp.float32),
    ]
    if emit_na:
        out_specs.append(row_spec)
        out_shape.append(jax.ShapeDtypeStruct((m, k), jnp.bfloat16))
    return pl.pallas_call(
        functools.partial(_dual_body, acc_t is not None, acc_s is not None,
                          c is None, emit_na),
        grid=(m // bm,),
        in_specs=in_specs,
        out_specs=out_specs,
        out_shape=out_shape,
        input_output_aliases=aliases,
        compiler_params=pltpu.CompilerParams(
            dimension_semantics=("arbitrary",)),
    )(*args)


def kernel(x_0, x_1, x_2, x_3, x_4, adjacency_0, adjacency_1, adjacency_2, adjacency_3, adjacency_4, cci_0_to_0, cci_1_to_1, cci_2_to_2, cci_3_to_3, cci_4_to_4, incidence_0_1, cci_0_to_1, incidence_0_2, cci_0_to_2, incidence_0_3, cci_0_to_3, incidence_0_4, cci_0_to_4, incidence_1_2, cci_1_to_2, incidence_1_3, cci_1_to_3, incidence_1_4, cci_1_to_4, incidence_2_3, cci_2_to_3, incidence_2_4, cci_2_to_4, incidence_3_4, cci_3_to_4, w_hbs_0_l1, w_hbs_4_l1, ws_hbns_0_1_l1, wt_hbns_0_1_l1, ws_hbns_1_2_l1, wt_hbns_1_2_l1, ws_hbns_2_3_l1, wt_hbns_2_3_l1, ws_hbns_3_4_l1, wt_hbns_3_4_l1, w_hbs_0_l2, w_hbs_1_l2, w_hbs_2_l2, w_hbs_3_l2, w_hbs_4_l2, ws_hbns_0_1_l2, wt_hbns_0_1_l2, ws_hbns_1_2_l2, wt_hbns_1_2_l2, ws_hbns_2_3_l2, wt_hbns_2_3_l2, ws_hbns_3_4_l2, wt_hbns_3_4_l2):
    h0_hbs, ft01 = _transform([x_0], [w_hbs_0_l1, wt_hbns_0_1_l1], [1.0, 1.0])
    fs01, ft12 = _transform([x_1], [ws_hbns_0_1_l1, wt_hbns_1_2_l1], [1.0, 1.0])
    fs12, ft23 = _transform([x_2], [ws_hbns_1_2_l1, wt_hbns_2_3_l1], [1.0, 1.0])
    fs23, ft34 = _transform([x_3], [ws_hbns_2_3_l1, wt_hbns_3_4_l1], [1.0, 1.0])
    h4_hbs, fs34 = _transform([x_4], [w_hbs_4_l1, ws_hbns_3_4_l1], [1.0, 1.0])

    x0, na0 = _hbs(adjacency_0, cci_0_to_0, h0_hbs, emit_na=True)
    (x0, x1, na01) = _dual(incidence_0_1, cci_0_to_1, fs01, ft01,
                           acc_t=x0, emit_na=True)
    (x1, x2, na12) = _dual(incidence_1_2, cci_1_to_2, fs12, ft12,
                           acc_t=x1, emit_na=True)
    (x2, x3, na23) = _dual(incidence_2_3, cci_2_to_3, fs23, ft23,
                           acc_t=x2, emit_na=True)
    (x3, x4, na34) = _dual(incidence_3_4, cci_3_to_4, fs34, ft34,
                           acc_t=x3, emit_na=True)
    x4, na4 = _hbs(adjacency_4, cci_4_to_4, h4_hbs, acc=x4, emit_na=True)

    k_agg = (2.0, 3.0, 3.0, 3.0, 2.0)

    def s(tgt):
        return 0.5 / k_agg[tgt]

    h0b, ft01b = _transform([x0], [w_hbs_0_l2, wt_hbns_0_1_l2], [s(0), s(1)])
    h1b, fs01b, ft12b = _transform(
        [x1], [w_hbs_1_l2, ws_hbns_0_1_l2, wt_hbns_1_2_l2], [s(1), s(0), s(2)])
    h2b, fs12b, ft23b = _transform(
        [x2], [w_hbs_2_l2, ws_hbns_1_2_l2, wt_hbns_2_3_l2], [s(2), s(1), s(3)])
    h3b, fs23b, ft34b = _transform(
        [x3], [w_hbs_3_l2, ws_hbns_2_3_l2, wt_hbns_3_4_l2], [s(3), s(2), s(4)])
    h4b, fs34b = _transform([x4], [w_hbs_4_l2, ws_hbns_3_4_l2], [s(4), s(3)])

    out0 = _hbs(na0, None, h0b)
    out1 = _hbs(adjacency_1, cci_1_to_1, h1b)
    out2 = _hbs(adjacency_2, cci_2_to_2, h2b)
    out3 = _hbs(adjacency_3, cci_3_to_3, h3b)
    out4 = _hbs(na4, None, h4b)
    out0, out1 = _dual(na01, None, fs01b, ft01b, acc_t=out0, acc_s=out1)
    out1, out2 = _dual(na12, None, fs12b, ft12b, acc_t=out1, acc_s=out2)
    out2, out3 = _dual(na23, None, fs23b, ft23b, acc_t=out2, acc_s=out3)
    out3, out4 = _dual(na34, None, fs34b, ft34b, acc_t=out3, acc_s=out4)
    return (out0, out1, out2, out3, out4)
